# baseline (speedup 1.0000x reference)
import functools

import jax
import jax.numpy as jnp
from jax import lax
from jax.experimental import pallas as pl
from jax.experimental.pallas import tpu as pltpu
from jax.experimental.pallas import tpu_sc as plsc

D = 768
BATCH = 4
SEQ = 8192
NC = 2
NS = 16
NW = NC * NS
SPW = SEQ // NW
C = 32
NCH = SPW // C
L = 16
VPR = D // L
UNR = 8

_mesh = plsc.VectorSubcoreMesh(core_axis_name="c", subcore_axis_name="s")

_ROUNDS = [(ch, b) for ch in range(NCH) for b in range(BATCH)]


@functools.partial(
    pl.kernel,
    mesh=_mesh,
    out_type=jax.ShapeDtypeStruct((BATCH * SEQ, D), jnp.float32),
    scratch_types=[
        pltpu.VMEM((BATCH * SPW,), jnp.int32),
        pltpu.VMEM((2, C, D), jnp.float32),
        pltpu.VMEM((2, C, D), jnp.float32),
        pltpu.SemaphoreType.DMA,
        pltpu.SemaphoreType.DMA,
        pltpu.SemaphoreType.DMA,
        pltpu.SemaphoreType.DMA,
        pltpu.SemaphoreType.DMA,
        pltpu.SemaphoreType.DMA,
    ],
)
def _embed(xf, tok, pos, out, idx_v, tokbuf, posb,
           gsem0, gsem1, ssem0, ssem1, psem0, psem1):
    wid = lax.axis_index("s") * NC + lax.axis_index("c")
    base_s = wid * SPW
    gsem = (gsem0, gsem1)
    ssem = (ssem0, ssem1)
    psem = (psem0, psem1)

    for b in range(BATCH):
        pltpu.sync_copy(xf.at[pl.ds(b * SEQ + base_s, SPW)],
                        idx_v.at[pl.ds(b * SPW, SPW)])

    def gather_start(r):
        ch, b = _ROUNDS[r]
        s = r % 2
        return pltpu.async_copy(
            tok.at[idx_v.at[pl.ds(b * SPW + ch * C, C)]],
            tokbuf.at[s], gsem[s])

    def pos_start(ch):
        return pltpu.async_copy(
            pos.at[pl.ds(base_s + ch * C, C)], posb.at[ch % 2], psem[ch % 2])

    R = NCH * BATCH
    ph = {0: pos_start(0)}
    gh = {0: gather_start(0)}
    sh = {}
    for r in range(R):
        ch, b = _ROUNDS[r]
        s = r % 2
        if r + 1 < R:
            if r - 1 >= 0:
                sh[r - 1].wait()
            gh[r + 1] = gather_start(r + 1)
        if b == 0:
            ph[ch].wait()
            if ch + 1 < NCH:
                ph[ch + 1] = pos_start(ch + 1)
        gh[r].wait()

        ps = ch % 2

        def row_body(rr, carry, _s=s, _ps=ps):
            def col_body(kk, carry2):
                for u in range(UNR):
                    sl = pl.ds(kk * (UNR * L) + u * L, L)
                    tokbuf[_s, rr, sl] = tokbuf[_s, rr, sl] + posb[_ps, rr, sl]
                return carry2
            lax.fori_loop(0, VPR // UNR, col_body, carry)
            return carry

        lax.fori_loop(0, C, row_body, 0)
        sh[r] = pltpu.async_copy(
            tokbuf.at[s], out.at[pl.ds(b * SEQ + base_s + ch * C, C)], ssem[s])
    sh[R - 2].wait()
    sh[R - 1].wait()


def kernel(x, token_emb, pos_emb):
    xf = x.reshape(-1).astype(jnp.int32)
    out = _embed(xf, token_emb, pos_emb)
    return out.reshape(BATCH, SEQ, D)


# speedup vs baseline: 2.8258x; 2.8258x over previous
import functools

import jax
import jax.numpy as jnp
from jax import lax
from jax.experimental import pallas as pl
from jax.experimental.pallas import tpu as pltpu
from jax.experimental.pallas import tpu_sc as plsc

D = 768
BATCH = 4
SEQ = 8192
NC = 2
NS = 16
NW = NC * NS
SPW = SEQ // NW
C = 32
NCH = SPW // C
L = 16
VPR = D // L

_mesh = plsc.VectorSubcoreMesh(core_axis_name="c", subcore_axis_name="s")


@functools.partial(
    pl.kernel,
    mesh=_mesh,
    out_type=jax.ShapeDtypeStruct((BATCH * SEQ, D), jnp.float32),
    scratch_types=[
        pltpu.VMEM((BATCH * SPW,), jnp.int32),
        pltpu.VMEM((2, C, D), jnp.float32),
        pltpu.VMEM((C, D), jnp.float32),
        pltpu.SemaphoreType.DMA,
        pltpu.SemaphoreType.DMA,
        pltpu.SemaphoreType.DMA,
        pltpu.SemaphoreType.DMA,
    ],
)
def _embed(xf, tok, pos, out, idx_v, tokbuf, posbuf,
           gsem0, gsem1, ssem0, ssem1):
    wid = lax.axis_index("s") * NC + lax.axis_index("c")
    base_s = wid * SPW
    gsem = (gsem0, gsem1)
    ssem = (ssem0, ssem1)

    for b in range(BATCH):
        pltpu.sync_copy(xf.at[pl.ds(b * SEQ + base_s, SPW)],
                        idx_v.at[pl.ds(b * SPW, SPW)])

    def gather_start(ch, b, slot):
        pltpu.async_copy(
            tok.at[idx_v.at[pl.ds(b * SPW + ch * C, C)]],
            tokbuf.at[slot], gsem[slot])

    def gather_drain(slot):
        pltpu.make_async_copy(
            tok.at[pl.ds(0, C)], tokbuf.at[slot], gsem[slot]).wait()

    def store_drain(slot):
        pltpu.make_async_copy(
            tokbuf.at[slot], out.at[pl.ds(0, C)], ssem[slot]).wait()

    gather_start(0, 0, 0)

    def chunk_body(ch, carry):
        pltpu.sync_copy(pos.at[pl.ds(base_s + ch * C, C)], posbuf)
        for b in range(BATCH):
            s = b % 2
            ns = 1 - s
            if b == 0:
                @pl.when(ch > 0)
                def _():
                    store_drain(ns)

                @pl.when(ch > 0)
                def _():
                    gather_start(ch, 1, ns)
                @pl.when(ch == 0)
                def _():
                    gather_start(0, 1, ns)
            else:
                store_drain(ns)
                if b < BATCH - 1:
                    gather_start(ch, b + 1, ns)
                else:
                    @pl.when(ch < NCH - 1)
                    def _():
                        gather_start(ch + 1, 0, ns)
            gather_drain(s)

            def row_body(rr, carry2, _s=s):
                for k in range(VPR):
                    sl = pl.ds(k * L, L)
                    tokbuf[_s, rr, sl] = tokbuf[_s, rr, sl] + posbuf[rr, sl]
                return carry2

            lax.fori_loop(0, C, row_body, 0)
            pltpu.async_copy(
                tokbuf.at[s],
                out.at[pl.ds(b * SEQ + base_s + ch * C, C)], ssem[s])
        return carry

    lax.fori_loop(0, NCH, chunk_body, 0)
    store_drain(1)


def kernel(x, token_emb, pos_emb):
    xf = x.reshape(-1).astype(jnp.int32)
    out = _embed(xf, token_emb, pos_emb)
    return out.reshape(BATCH, SEQ, D)
